# Initial kernel scaffold; baseline (speedup 1.0000x reference)
import jax
import jax.numpy as jnp
from jax.experimental import pallas as pl


def kernel(vertices, op_values, op_rows, op_cols):
    raise NotImplementedError("write your pallas kernel here")


# speedup vs baseline: 2.8452x; 2.8452x over previous
import jax
import jax.numpy as jnp
from jax import lax
from jax.experimental import pallas as pl
from jax.experimental.pallas import tpu as pltpu
from jax.experimental.pallas import tpu_sc as plsc

_N_HD = 20000
_N_SMPL = 6890
_N_SMPL_PAD = 6912
_B = 256
_NC = 2
_NS = 16
_NW = _NC * _NS
_B_PER_W = _B // _NW
_R = 2000
_NCH = _N_HD // _R
_GRP = _R // 16

_XSCALE = 2.0 / 0.15


def _sc_body(y_hbm, c0_hbm, c1_hbm, c2_hbm, w0_hbm, w1_hbm, w2_hbm, out_hbm,
             table, cb0, cb1, cb2, wb0, wb1, wb2, outbuf):
    wid = lax.axis_index("s") * _NC + lax.axis_index("c")
    b0 = wid * _B_PER_W
    pltpu.sync_copy(y_hbm.at[pl.ds(b0 * _N_SMPL_PAD, _B_PER_W * _N_SMPL_PAD)],
                    table)

    for ch in range(_NCH):
        off = ch * _R
        pltpu.sync_copy(c0_hbm.at[pl.ds(off, _R)], cb0)
        pltpu.sync_copy(c1_hbm.at[pl.ds(off, _R)], cb1)
        pltpu.sync_copy(c2_hbm.at[pl.ds(off, _R)], cb2)
        pltpu.sync_copy(w0_hbm.at[pl.ds(off, _R)], wb0)
        pltpu.sync_copy(w1_hbm.at[pl.ds(off, _R)], wb1)
        pltpu.sync_copy(w2_hbm.at[pl.ds(off, _R)], wb2)

        @pl.loop(0, _GRP)
        def _group(g):
            base = g * 16
            c0 = cb0[pl.ds(base, 16)]
            c1 = cb1[pl.ds(base, 16)]
            c2 = cb2[pl.ds(base, 16)]
            w0 = wb0[pl.ds(base, 16)]
            w1 = wb1[pl.ds(base, 16)]
            w2 = wb2[pl.ds(base, 16)]
            for b in range(_B_PER_W):
                boff = b * _N_SMPL_PAD
                g0 = plsc.load_gather(table, [c0 + boff])
                g1 = plsc.load_gather(table, [c1 + boff])
                g2 = plsc.load_gather(table, [c2 + boff])
                h = g0 * w0 + g1 * w1 + g2 * w2
                e = jnp.exp(h * _XSCALE)
                t = 1.0 - 2.0 / (e + 1.0)
                outbuf[pl.ds(b * _R + base, 16)] = (
                    jnp.where(h < 0.0, 10.0, 1.0) * (t * t))

        for b in range(_B_PER_W):
            pltpu.sync_copy(outbuf.at[pl.ds(b * _R, _R)],
                            out_hbm.at[pl.ds((b0 + b) * _N_HD + off, _R)])


@jax.jit
def kernel(vertices, op_values, op_rows, op_cols):
    del op_rows
    y = vertices[:, :, 1]
    y_pad = jnp.zeros((_B, _N_SMPL_PAD), jnp.float32).at[:, :_N_SMPL].set(y)
    cols = op_cols.astype(jnp.int32).reshape(_N_HD, 3)
    vals = op_values.astype(jnp.float32).reshape(_N_HD, 3)

    mesh = plsc.VectorSubcoreMesh(core_axis_name="c", subcore_axis_name="s")
    fn = pl.kernel(
        _sc_body,
        out_type=jax.ShapeDtypeStruct((_B * _N_HD,), jnp.float32),
        mesh=mesh,
        compiler_params=pltpu.CompilerParams(
            use_tc_tiling_on_sc=False, needs_layout_passes=False),
        scratch_types=[
            pltpu.VMEM((_B_PER_W * _N_SMPL_PAD,), jnp.float32),
            pltpu.VMEM((_R,), jnp.int32),
            pltpu.VMEM((_R,), jnp.int32),
            pltpu.VMEM((_R,), jnp.int32),
            pltpu.VMEM((_R,), jnp.float32),
            pltpu.VMEM((_R,), jnp.float32),
            pltpu.VMEM((_R,), jnp.float32),
            pltpu.VMEM((_B_PER_W * _R,), jnp.float32),
        ],
    )
    out_flat = fn(y_pad.reshape(-1), cols[:, 0], cols[:, 1], cols[:, 2],
                  vals[:, 0], vals[:, 1], vals[:, 2])
    return out_flat.reshape(_B, _N_HD)


# speedup vs baseline: 3.1660x; 1.1127x over previous
import jax
import jax.numpy as jnp
from jax import lax
from jax.experimental import pallas as pl
from jax.experimental.pallas import tpu as pltpu
from jax.experimental.pallas import tpu_sc as plsc

_N_HD = 20000
_N_SMPL = 6890
_N_SMPL_PAD = 6912
_B = 256
_NC = 2
_NS = 16
_NW = _NC * _NS
_B_PER_W = _B // _NW
_R = 2000
_NCH = _N_HD // _R
_GRP = _R // 16

_XSCALE = 2.0 / 0.15


def _sc_body(y_hbm, c0_hbm, c1_hbm, c2_hbm, w0_hbm, w1_hbm, w2_hbm, out_hbm,
             table, cb0, cb1, cb2, wb0, wb1, wb2, ob,
             si0, si1, so0, so1):
    wid = lax.axis_index("s") * _NC + lax.axis_index("c")
    b0 = wid * _B_PER_W

    in_refs = (c0_hbm, c1_hbm, c2_hbm, w0_hbm, w1_hbm, w2_hbm)
    in_bufs = (cb0, cb1, cb2, wb0, wb1, wb2)
    sems_in = (si0, si1)
    sems_out = (so0, so1)

    def fire_in(ch):
        par = ch % 2
        off = ch * _R
        return [pltpu.async_copy(r.at[pl.ds(off, _R)], buf.at[par],
                                 sems_in[par])
                for r, buf in zip(in_refs, in_bufs)]

    pend_in = {0: fire_in(0)}
    pltpu.sync_copy(y_hbm.at[pl.ds(b0 * _N_SMPL_PAD, _B_PER_W * _N_SMPL_PAD)],
                    table)
    pend_out = {}

    for ch in range(_NCH):
        par = ch % 2
        if ch + 1 < _NCH:
            pend_in[ch + 1] = fire_in(ch + 1)
        for d in pend_in.pop(ch):
            d.wait()
        if ch - 2 in pend_out:
            for d in pend_out.pop(ch - 2):
                d.wait()

        @pl.loop(0, _GRP, unroll=2)
        def _group(g):
            base = g * 16
            c0 = cb0[par, pl.ds(base, 16)]
            c1 = cb1[par, pl.ds(base, 16)]
            c2 = cb2[par, pl.ds(base, 16)]
            w0 = wb0[par, pl.ds(base, 16)]
            w1 = wb1[par, pl.ds(base, 16)]
            w2 = wb2[par, pl.ds(base, 16)]
            for b in range(_B_PER_W):
                boff = b * _N_SMPL_PAD
                g0 = plsc.load_gather(table, [c0 + boff])
                g1 = plsc.load_gather(table, [c1 + boff])
                g2 = plsc.load_gather(table, [c2 + boff])
                h = g0 * w0 + g1 * w1 + g2 * w2
                e = jnp.exp(h * _XSCALE)
                t = 1.0 - 2.0 / (e + 1.0)
                ob[par, pl.ds(b * _R + base, 16)] = (
                    jnp.where(h < 0.0, 10.0, 1.0) * (t * t))

        off = ch * _R
        pend_out[ch] = [
            pltpu.async_copy(ob.at[par, pl.ds(b * _R, _R)],
                             out_hbm.at[pl.ds((b0 + b) * _N_HD + off, _R)],
                             sems_out[par])
            for b in range(_B_PER_W)]

    for ch in sorted(pend_out):
        for d in pend_out[ch]:
            d.wait()


@jax.jit
def kernel(vertices, op_values, op_rows, op_cols):
    del op_rows
    y = vertices[:, :, 1]
    y_pad = jnp.zeros((_B, _N_SMPL_PAD), jnp.float32).at[:, :_N_SMPL].set(y)
    cols = op_cols.astype(jnp.int32).reshape(_N_HD, 3)
    vals = op_values.astype(jnp.float32).reshape(_N_HD, 3)

    mesh = plsc.VectorSubcoreMesh(core_axis_name="c", subcore_axis_name="s")
    fn = pl.kernel(
        _sc_body,
        out_type=jax.ShapeDtypeStruct((_B * _N_HD,), jnp.float32),
        mesh=mesh,
        compiler_params=pltpu.CompilerParams(
            use_tc_tiling_on_sc=False, needs_layout_passes=False),
        scratch_types=[
            pltpu.VMEM((_B_PER_W * _N_SMPL_PAD,), jnp.float32),
            pltpu.VMEM((2, _R), jnp.int32),
            pltpu.VMEM((2, _R), jnp.int32),
            pltpu.VMEM((2, _R), jnp.int32),
            pltpu.VMEM((2, _R), jnp.float32),
            pltpu.VMEM((2, _R), jnp.float32),
            pltpu.VMEM((2, _R), jnp.float32),
            pltpu.VMEM((2, _B_PER_W * _R), jnp.float32),
            pltpu.SemaphoreType.DMA,
            pltpu.SemaphoreType.DMA,
            pltpu.SemaphoreType.DMA,
            pltpu.SemaphoreType.DMA,
        ],
    )
    out_flat = fn(y_pad.reshape(-1), cols[:, 0], cols[:, 1], cols[:, 2],
                  vals[:, 0], vals[:, 1], vals[:, 2])
    return out_flat.reshape(_B, _N_HD)


# speedup vs baseline: 5.4358x; 1.7169x over previous
import jax
import jax.numpy as jnp
from jax import lax
from jax.experimental import pallas as pl
from jax.experimental.pallas import tpu as pltpu
from jax.experimental.pallas import tpu_sc as plsc

_N_HD = 20000
_N_SMPL = 6890
_B = 256
_NC = 2
_NS = 16
_NW = _NC * _NS
_B_PER_W = _B // _NW
_R = 2000
_NCH = _N_HD // _R
_GRP = _R // 16

_INV_A2 = 1.0 / 0.15


def _sc_body(y_hbm, c0_hbm, c1_hbm, c2_hbm, w0_hbm, w1_hbm, w2_hbm, h_hbm,
             table, cb0, cb1, cb2, wb0, wb1, wb2, ob,
             si0, si1, so0, so1):
    wid = lax.axis_index("s") * _NC + lax.axis_index("c")
    b0 = wid * _B_PER_W

    in_refs = (c0_hbm, c1_hbm, c2_hbm, w0_hbm, w1_hbm, w2_hbm)
    in_bufs = (cb0, cb1, cb2, wb0, wb1, wb2)
    sems_in = (si0, si1)
    sems_out = (so0, so1)

    def fire_in(ch):
        par = ch % 2
        off = ch * _R
        return [pltpu.async_copy(r.at[pl.ds(off, _R)], buf.at[par],
                                 sems_in[par])
                for r, buf in zip(in_refs, in_bufs)]

    pend_in = {0: fire_in(0)}
    pltpu.sync_copy(y_hbm.at[pl.ds(b0 * _N_SMPL, _B_PER_W * _N_SMPL)], table)
    pend_out = {}

    for ch in range(_NCH):
        par = ch % 2
        if ch + 1 < _NCH:
            pend_in[ch + 1] = fire_in(ch + 1)
        for d in pend_in.pop(ch):
            d.wait()
        if ch - 2 in pend_out:
            for d in pend_out.pop(ch - 2):
                d.wait()

        @pl.loop(0, _GRP, unroll=2)
        def _group(g):
            base = g * 16
            c0 = cb0[par, pl.ds(base, 16)]
            c1 = cb1[par, pl.ds(base, 16)]
            c2 = cb2[par, pl.ds(base, 16)]
            w0 = wb0[par, pl.ds(base, 16)]
            w1 = wb1[par, pl.ds(base, 16)]
            w2 = wb2[par, pl.ds(base, 16)]
            for b in range(_B_PER_W):
                boff = b * _N_SMPL
                g0 = plsc.load_gather(table, [c0 + boff])
                g1 = plsc.load_gather(table, [c1 + boff])
                g2 = plsc.load_gather(table, [c2 + boff])
                ob[par, pl.ds(b * _R + base, 16)] = g0 * w0 + g1 * w1 + g2 * w2

        off = ch * _R
        pend_out[ch] = [
            pltpu.async_copy(ob.at[par, pl.ds(b * _R, _R)],
                             h_hbm.at[pl.ds((b0 + b) * _N_HD + off, _R)],
                             sems_out[par])
            for b in range(_B_PER_W)]

    for ch in sorted(pend_out):
        for d in pend_out[ch]:
            d.wait()


def _tc_body(h_ref, o_ref):
    h = h_ref[...]
    t = jnp.tanh(h * _INV_A2)
    o_ref[...] = jnp.where(h < 0.0, 10.0, 1.0) * (t * t)


@jax.jit
def kernel(vertices, op_values, op_rows, op_cols):
    del op_rows
    y = vertices[:, :, 1]
    cols = op_cols.astype(jnp.int32).reshape(_N_HD, 3)
    vals = op_values.astype(jnp.float32).reshape(_N_HD, 3)

    mesh = plsc.VectorSubcoreMesh(core_axis_name="c", subcore_axis_name="s")
    sc_fn = pl.kernel(
        _sc_body,
        out_type=jax.ShapeDtypeStruct((_B * _N_HD,), jnp.float32),
        mesh=mesh,
        compiler_params=pltpu.CompilerParams(
            use_tc_tiling_on_sc=False, needs_layout_passes=False),
        scratch_types=[
            pltpu.VMEM((_B_PER_W * _N_SMPL,), jnp.float32),
            pltpu.VMEM((2, _R), jnp.int32),
            pltpu.VMEM((2, _R), jnp.int32),
            pltpu.VMEM((2, _R), jnp.int32),
            pltpu.VMEM((2, _R), jnp.float32),
            pltpu.VMEM((2, _R), jnp.float32),
            pltpu.VMEM((2, _R), jnp.float32),
            pltpu.VMEM((2, _B_PER_W * _R), jnp.float32),
            pltpu.SemaphoreType.DMA,
            pltpu.SemaphoreType.DMA,
            pltpu.SemaphoreType.DMA,
            pltpu.SemaphoreType.DMA,
        ],
    )
    h_flat = sc_fn(y.reshape(-1), cols[:, 0], cols[:, 1], cols[:, 2],
                   vals[:, 0], vals[:, 1], vals[:, 2])
    h = h_flat.reshape(_B, _N_HD)

    blk = 32
    out = pl.pallas_call(
        _tc_body,
        out_shape=jax.ShapeDtypeStruct((_B, _N_HD), jnp.float32),
        grid=(_B // blk,),
        in_specs=[pl.BlockSpec((blk, _N_HD), lambda i: (i, 0))],
        out_specs=pl.BlockSpec((blk, _N_HD), lambda i: (i, 0)),
    )(h)
    return out
